# baseline (speedup 1.0000x reference)
import jax
import jax.numpy as jnp
from jax import lax
from jax.experimental import pallas as pl
from jax.experimental.pallas import tpu as pltpu
from jax.experimental.pallas import tpu_sc as plsc

_M = 64
_C = 200
_BW = 128
_PC = 8
_NCK = _C // _PC


def _positional_embedding(num_positions):
    pos = jnp.arange(num_positions)
    denom = 10000 ** jnp.linspace(0, 1, _M)
    arg = jnp.expand_dims(pos, 1) / jnp.expand_dims(denom, 0)
    pos_emb = jnp.zeros((num_positions, _M), jnp.float32)
    pos_emb = pos_emb.at[:, ::2].set(jnp.sin(arg[:, ::2]))
    pos_emb = pos_emb.at[:, 1::2].set(jnp.cos(arg[:, 1::2]))
    return pos_emb


def kernel(input, emb):
    B, C = input.shape
    assert C == _C and emb.shape[1] == _M

    info = plsc.get_sparse_core_info()
    NC, NS = info.num_cores, info.num_subcores
    NW = NC * NS
    assert B % NW == 0 and B // NW == _BW

    idx_t = input.T
    emb_t = emb.T
    T = emb.shape[0]
    NPAIR = T // 2
    NBLK = T // 128
    TAIL = T - NBLK * 128
    pos = _positional_embedding(C)
    posx = jnp.broadcast_to(pos[:, :, None], (_C, _M, 16)).reshape(-1)

    mesh = plsc.VectorSubcoreMesh(core_axis_name="c", subcore_axis_name="s")

    @pl.kernel(
        out_type=jax.ShapeDtypeStruct((NPAIR, 2 * _M), jnp.float32),
        mesh=mesh,
        compiler_params=pltpu.CompilerParams(use_tc_tiling_on_sc=True,
                                             needs_layout_passes=False),
        scratch_types=[
            pltpu.VMEM((2, _M, 128), jnp.float32),
            pltpu.VMEM((2, _M, 128), jnp.float32),
            pltpu.SemaphoreType.DMA,
            pltpu.SemaphoreType.DMA,
            pltpu.SemaphoreType.DMA,
            pltpu.SemaphoreType.DMA,
        ],
    )
    def stage(embt_hbm, tail_hbm, pairs_hbm, x_v, y_v, i0, i1, o0, o1):
        isem = (i0, i1)
        osem = (o0, o1)
        w = lax.axis_index("s") * NC + lax.axis_index("c")
        iotas4 = [lax.iota(jnp.int32, 16) + 16 * q for q in range(4)]

        def t_of(n):
            return w + NW * n

        def in_start(n, k):
            pltpu.async_copy(embt_hbm.at[:, pl.ds(t_of(n) * 128, 128)],
                             x_v.at[k], isem[k])

        def in_wait(k):
            pltpu.make_async_copy(embt_hbm.at[:, pl.ds(0, 128)], x_v.at[k],
                                  isem[k]).wait()

        def out_start(n, k):
            pltpu.async_copy(y_v.at[k],
                             pairs_hbm.at[pl.ds(t_of(n) * 64, 64)], osem[k])

        def out_wait(k):
            pltpu.make_async_copy(y_v.at[k], pairs_hbm.at[pl.ds(0, 64)],
                                  osem[k]).wait()

        def transpose_blk(k):
            @plsc.parallel_loop(0, _M, unroll=4)
            def _(r):
                c0 = jnp.broadcast_to(2 * r, (16,))
                c1 = c0 + 1
                for q in range(8):
                    col = c0 if q < 4 else c1
                    val = plsc.load_gather(x_v.at[k], [iotas4[q % 4], col])
                    y_v[k, r, pl.ds(16 * q, 16)] = val

        nfull = NBLK // NW
        in_start(0, 0)
        in_start(1, 1)

        @pl.loop(0, nfull, step=2)
        def _(n):
            for k in range(2):
                j = n + k
                in_wait(k)

                @pl.when(j >= 2)
                def _():
                    out_wait(k)

                transpose_blk(k)
                out_start(j, k)

                @pl.when(t_of(j + 2) < NBLK)
                def _():
                    in_start(j + 2, k)

        extra = NBLK - nfull * NW
        if extra:
            @pl.when(w < extra)
            def _():
                in_wait(0)
                out_wait(0)
                transpose_blk(0)
                out_start(nfull, 0)

        out_wait(0)
        out_wait(1)

        if TAIL:
            @pl.when(w == 0)
            def _():
                pltpu.sync_copy(tail_hbm, y_v.at[0, pl.ds(0, TAIL // 2)])
                pltpu.sync_copy(y_v.at[0, pl.ds(0, TAIL // 2)],
                                pairs_hbm.at[pl.ds(NBLK * 64, TAIL // 2)])

    @pl.kernel(
        out_type=jax.ShapeDtypeStruct((_C, _M, B), jnp.float32),
        mesh=mesh,
        compiler_params=pltpu.CompilerParams(use_tc_tiling_on_sc=True,
                                             needs_layout_passes=False),
        scratch_types=[
            pltpu.VMEM((2, _PC, _BW), jnp.int32),
            pltpu.VMEM((2, _PC, _BW), jnp.int32),
            pltpu.VMEM((2, _PC, _BW), jnp.int32),
            pltpu.VMEM((2, _BW, 2 * _M), jnp.float32),
            pltpu.VMEM((2, _M, _BW), jnp.float32),
            pltpu.VMEM((_PC * _M * 16,), jnp.float32),
            pltpu.VMEM((_PC * _M * 16,), jnp.float32),
            pltpu.SemaphoreType.DMA,
            pltpu.SemaphoreType.DMA,
            pltpu.SemaphoreType.DMA,
            pltpu.SemaphoreType.DMA,
            pltpu.SemaphoreType.DMA,
            pltpu.SemaphoreType.DMA,
        ],
    )
    def run(idx_hbm, tab_hbm, posx_hbm, out_hbm, idx_v, pidx_v, hv64_v,
            rows_v, t_v, posx0_v, posx1_v, isem, psem, gsem0, gsem1,
            osem0, osem1):
        gsem = (gsem0, gsem1)
        osem = (osem0, osem1)
        posx_v = (posx0_v, posx1_v)
        w = lax.axis_index("s") * NC + lax.axis_index("c")
        bw = w * _BW
        iotas = [lax.iota(jnp.int32, 16) + 16 * g for g in range(_BW // 16)]

        def idx_fetch(ck, s):
            pltpu.async_copy(idx_hbm.at[pl.ds(ck * _PC, _PC), pl.ds(bw, _BW)],
                             idx_v.at[s], isem)

        def idx_wait(s):
            pltpu.make_async_copy(idx_hbm.at[pl.ds(0, _PC), pl.ds(0, _BW)],
                                  idx_v.at[s], isem).wait()

        def posx_fetch(ck, s):
            pltpu.async_copy(posx_hbm.at[pl.ds(ck * _PC * _M * 16,
                                               _PC * _M * 16)],
                             posx_v[s], psem)

        def posx_wait(s):
            pltpu.make_async_copy(posx_hbm.at[pl.ds(0, _PC * _M * 16)],
                                  posx_v[s], psem).wait()

        def derive(s):
            for p in range(_PC):
                for g in range(_BW // 16):
                    v = idx_v[s, p, pl.ds(16 * g, 16)]
                    pidx_v[s, p, pl.ds(16 * g, 16)] = v >> 1
                    hv64_v[s, p, pl.ds(16 * g, 16)] = (v & 1) << 6

        def gather_start(s, p, b):
            pltpu.async_copy(tab_hbm.at[pidx_v.at[s, p]], rows_v.at[b],
                             gsem[b])

        def gather_wait(b):
            pltpu.make_async_copy(tab_hbm.at[pl.ds(0, _BW)], rows_v.at[b],
                                  gsem[b]).wait()

        def out_start(c, b):
            pltpu.async_copy(t_v.at[b], out_hbm.at[c, :, pl.ds(bw, _BW)],
                             osem[b])

        def out_wait(b):
            pltpu.make_async_copy(t_v.at[b], out_hbm.at[0, :, pl.ds(0, _BW)],
                                  osem[b]).wait()

        def transpose_add(s, p, b):
            hv = [hv64_v[s, p, pl.ds(16 * g, 16)] for g in range(_BW // 16)]

            @plsc.parallel_loop(0, _M, unroll=8)
            def _(f):
                pv = posx_v[s][pl.ds((p * _M + f) * 16, 16)]
                for g in range(_BW // 16):
                    val = plsc.load_gather(rows_v.at[b], [iotas[g], hv[g] + f])
                    t_v[b, f, pl.ds(16 * g, 16)] = val + pv

        idx_fetch(0, 0)
        posx_fetch(0, 0)
        idx_wait(0)
        posx_wait(0)
        derive(0)
        idx_fetch(1, 1)
        posx_fetch(1, 1)
        gather_start(0, 0, 0)
        gather_start(0, 1, 1)

        def chunk_body(ct, cb):
            @pl.when(ct + 1 < _NCK)
            def _():
                idx_wait(1 - cb)
                posx_wait(1 - cb)
                derive(1 - cb)

            for p in range(_PC):
                c = ct * _PC + p
                b = p % 2
                gather_wait(b)

                @pl.when(c >= 2)
                def _():
                    out_wait(b)

                transpose_add(cb, p, b)
                out_start(c, b)
                if p < _PC - 2:
                    gather_start(cb, p + 2, b)
                else:
                    @pl.when(ct + 1 < _NCK)
                    def _():
                        gather_start(1 - cb, p + 2 - _PC, b)

            @pl.when(ct + 2 < _NCK)
            def _():
                idx_fetch(ct + 2, cb)
                posx_fetch(ct + 2, cb)

        @pl.loop(0, _NCK - 1, step=2)
        def _(t):
            chunk_body(t, 0)
            chunk_body(t + 1, 1)

        chunk_body(_NCK - 1, 0)
        out_wait(0)
        out_wait(1)

    tail_rows = emb[NBLK * 128:].reshape(TAIL // 2, 2 * _M) if TAIL else None
    table = stage(emb_t, tail_rows)
    out = run(idx_t, table, posx)
    return out.transpose(2, 0, 1)


# speedup vs baseline: 1.0002x; 1.0002x over previous
import jax
import jax.numpy as jnp
from jax import lax
from jax.experimental import pallas as pl
from jax.experimental.pallas import tpu as pltpu
from jax.experimental.pallas import tpu_sc as plsc

_M = 64
_C = 200
_BW = 128
_PC = 8
_NCK = _C // _PC


def _positional_embedding(num_positions):
    pos = jnp.arange(num_positions)
    denom = 10000 ** jnp.linspace(0, 1, _M)
    arg = jnp.expand_dims(pos, 1) / jnp.expand_dims(denom, 0)
    pos_emb = jnp.zeros((num_positions, _M), jnp.float32)
    pos_emb = pos_emb.at[:, ::2].set(jnp.sin(arg[:, ::2]))
    pos_emb = pos_emb.at[:, 1::2].set(jnp.cos(arg[:, 1::2]))
    return pos_emb


def kernel(input, emb):
    B, C = input.shape
    assert C == _C and emb.shape[1] == _M

    info = plsc.get_sparse_core_info()
    NC, NS = info.num_cores, info.num_subcores
    NW = NC * NS
    assert B % NW == 0 and B // NW == _BW

    idx_t = input.T
    emb_t = emb.T
    T = emb.shape[0]
    NPAIR = T // 2
    NBLK = T // 128
    TAIL = T - NBLK * 128
    pos = _positional_embedding(C)
    posx = jnp.broadcast_to(pos[:, :, None], (_C, _M, 16)).reshape(-1)

    mesh = plsc.VectorSubcoreMesh(core_axis_name="c", subcore_axis_name="s")

    @pl.kernel(
        out_type=jax.ShapeDtypeStruct((NPAIR, 2 * _M), jnp.float32),
        mesh=mesh,
        compiler_params=pltpu.CompilerParams(use_tc_tiling_on_sc=True,
                                             needs_layout_passes=False),
        scratch_types=[
            pltpu.VMEM((4, _M, 128), jnp.float32),
            pltpu.VMEM((4, _M, 128), jnp.float32),
            pltpu.SemaphoreType.DMA,
            pltpu.SemaphoreType.DMA,
            pltpu.SemaphoreType.DMA,
            pltpu.SemaphoreType.DMA,
            pltpu.SemaphoreType.DMA,
            pltpu.SemaphoreType.DMA,
            pltpu.SemaphoreType.DMA,
            pltpu.SemaphoreType.DMA,
        ],
    )
    def stage(embt_hbm, tail_hbm, pairs_hbm, x_v, y_v, i0, i1, i2, i3,
              o0, o1, o2, o3):
        isem = (i0, i1, i2, i3)
        osem = (o0, o1, o2, o3)
        w = lax.axis_index("s") * NC + lax.axis_index("c")
        iotas4 = [lax.iota(jnp.int32, 16) + 16 * q for q in range(4)]

        def t_of(n):
            return w + NW * n

        def in_start(n, k):
            pltpu.async_copy(embt_hbm.at[:, pl.ds(t_of(n) * 128, 128)],
                             x_v.at[k], isem[k])

        def in_wait(k):
            pltpu.make_async_copy(embt_hbm.at[:, pl.ds(0, 128)], x_v.at[k],
                                  isem[k]).wait()

        def out_start(n, k):
            pltpu.async_copy(y_v.at[k],
                             pairs_hbm.at[pl.ds(t_of(n) * 64, 64)], osem[k])

        def out_wait(k):
            pltpu.make_async_copy(y_v.at[k], pairs_hbm.at[pl.ds(0, 64)],
                                  osem[k]).wait()

        def transpose_blk(k):
            @plsc.parallel_loop(0, _M, unroll=4)
            def _(r):
                c0 = jnp.broadcast_to(2 * r, (16,))
                c1 = c0 + 1
                for q in range(8):
                    col = c0 if q < 4 else c1
                    val = plsc.load_gather(x_v.at[k], [iotas4[q % 4], col])
                    y_v[k, r, pl.ds(16 * q, 16)] = val

        nfull = NBLK // NW
        for k0 in range(4):
            in_start(k0, k0)

        @pl.loop(0, nfull, step=4)
        def _(n):
            for k in range(4):
                j = n + k
                in_wait(k)

                @pl.when(j >= 4)
                def _():
                    out_wait(k)

                transpose_blk(k)
                out_start(j, k)

                @pl.when(t_of(j + 4) < NBLK)
                def _():
                    in_start(j + 4, k)

        extra = NBLK - nfull * NW
        if extra:
            @pl.when(w < extra)
            def _():
                in_wait(0)
                out_wait(0)
                transpose_blk(0)
                out_start(nfull, 0)

        out_wait(0)
        out_wait(1)
        out_wait(2)
        out_wait(3)

        if TAIL:
            @pl.when(w == 0)
            def _():
                pltpu.sync_copy(tail_hbm, y_v.at[0, pl.ds(0, TAIL // 2)])
                pltpu.sync_copy(y_v.at[0, pl.ds(0, TAIL // 2)],
                                pairs_hbm.at[pl.ds(NBLK * 64, TAIL // 2)])

    @pl.kernel(
        out_type=jax.ShapeDtypeStruct((_C, _M, B), jnp.float32),
        mesh=mesh,
        compiler_params=pltpu.CompilerParams(use_tc_tiling_on_sc=True,
                                             needs_layout_passes=False),
        scratch_types=[
            pltpu.VMEM((2, _PC, _BW), jnp.int32),
            pltpu.VMEM((2, _PC, _BW), jnp.int32),
            pltpu.VMEM((2, _PC, _BW), jnp.int32),
            pltpu.VMEM((4, _BW, 2 * _M), jnp.float32),
            pltpu.VMEM((2, _M, _BW), jnp.float32),
            pltpu.VMEM((_PC * _M * 16,), jnp.float32),
            pltpu.VMEM((_PC * _M * 16,), jnp.float32),
            pltpu.SemaphoreType.DMA,
            pltpu.SemaphoreType.DMA,
            pltpu.SemaphoreType.DMA,
            pltpu.SemaphoreType.DMA,
            pltpu.SemaphoreType.DMA,
            pltpu.SemaphoreType.DMA,
            pltpu.SemaphoreType.DMA,
            pltpu.SemaphoreType.DMA,
        ],
    )
    def run(idx_hbm, tab_hbm, posx_hbm, out_hbm, idx_v, pidx_v, hv64_v,
            rows_v, t_v, posx0_v, posx1_v, isem, psem, gsem0, gsem1,
            gsem2, gsem3, osem0, osem1):
        gsem = (gsem0, gsem1, gsem2, gsem3)
        osem = (osem0, osem1)
        posx_v = (posx0_v, posx1_v)
        w = lax.axis_index("s") * NC + lax.axis_index("c")
        bw = w * _BW
        iotas = [lax.iota(jnp.int32, 16) + 16 * g for g in range(_BW // 16)]

        def idx_fetch(ck, s):
            pltpu.async_copy(idx_hbm.at[pl.ds(ck * _PC, _PC), pl.ds(bw, _BW)],
                             idx_v.at[s], isem)

        def idx_wait(s):
            pltpu.make_async_copy(idx_hbm.at[pl.ds(0, _PC), pl.ds(0, _BW)],
                                  idx_v.at[s], isem).wait()

        def posx_fetch(ck, s):
            pltpu.async_copy(posx_hbm.at[pl.ds(ck * _PC * _M * 16,
                                               _PC * _M * 16)],
                             posx_v[s], psem)

        def posx_wait(s):
            pltpu.make_async_copy(posx_hbm.at[pl.ds(0, _PC * _M * 16)],
                                  posx_v[s], psem).wait()

        def derive(s):
            for p in range(_PC):
                for g in range(_BW // 16):
                    v = idx_v[s, p, pl.ds(16 * g, 16)]
                    pidx_v[s, p, pl.ds(16 * g, 16)] = v >> 1
                    hv64_v[s, p, pl.ds(16 * g, 16)] = (v & 1) << 6

        def gather_start(s, p, b):
            pltpu.async_copy(tab_hbm.at[pidx_v.at[s, p]], rows_v.at[b],
                             gsem[b])

        def gather_wait(b):
            pltpu.make_async_copy(tab_hbm.at[pl.ds(0, _BW)], rows_v.at[b],
                                  gsem[b]).wait()

        def out_start(c, b):
            pltpu.async_copy(t_v.at[b], out_hbm.at[c, :, pl.ds(bw, _BW)],
                             osem[b])

        def out_wait(b):
            pltpu.make_async_copy(t_v.at[b], out_hbm.at[0, :, pl.ds(0, _BW)],
                                  osem[b]).wait()

        def transpose_add(s, p, b, b2):
            hv = [hv64_v[s, p, pl.ds(16 * g, 16)] for g in range(_BW // 16)]

            @plsc.parallel_loop(0, _M, unroll=8)
            def _(f):
                pv = posx_v[s][pl.ds((p * _M + f) * 16, 16)]
                for g in range(_BW // 16):
                    val = plsc.load_gather(rows_v.at[b], [iotas[g], hv[g] + f])
                    t_v[b2, f, pl.ds(16 * g, 16)] = val + pv

        idx_fetch(0, 0)
        posx_fetch(0, 0)
        idx_wait(0)
        posx_wait(0)
        derive(0)
        idx_fetch(1, 1)
        posx_fetch(1, 1)
        for p0 in range(4):
            gather_start(0, p0, p0)

        def chunk_body(ct, cb):
            @pl.when(ct + 1 < _NCK)
            def _():
                idx_wait(1 - cb)
                posx_wait(1 - cb)
                derive(1 - cb)

            for p in range(_PC):
                c = ct * _PC + p
                b = p % 4
                b2 = p % 2
                gather_wait(b)

                @pl.when(c >= 2)
                def _():
                    out_wait(b2)

                transpose_add(cb, p, b, b2)
                out_start(c, b2)
                if p < _PC - 4:
                    gather_start(cb, p + 4, b)
                else:
                    @pl.when(ct + 1 < _NCK)
                    def _():
                        gather_start(1 - cb, p + 4 - _PC, b)

            @pl.when(ct + 2 < _NCK)
            def _():
                idx_fetch(ct + 2, cb)
                posx_fetch(ct + 2, cb)

        @pl.loop(0, _NCK - 1, step=2)
        def _(t):
            chunk_body(t, 0)
            chunk_body(t + 1, 1)

        chunk_body(_NCK - 1, 0)
        out_wait(0)
        out_wait(1)

    tail_rows = emb[NBLK * 128:].reshape(TAIL // 2, 2 * _M) if TAIL else None
    table = stage(emb_t, tail_rows)
    out = run(idx_t, table, posx)
    return out.transpose(2, 0, 1)


# speedup vs baseline: 1.1991x; 1.1989x over previous
import jax
import jax.numpy as jnp
from jax import lax
from jax.experimental import pallas as pl
from jax.experimental.pallas import tpu as pltpu
from jax.experimental.pallas import tpu_sc as plsc

_M = 64
_C = 200
_BW = 128
_MC = 4
_G = _C // _MC


def _positional_embedding(num_positions):
    pos = jnp.arange(num_positions)
    denom = 10000 ** jnp.linspace(0, 1, _M)
    arg = jnp.expand_dims(pos, 1) / jnp.expand_dims(denom, 0)
    pos_emb = jnp.zeros((num_positions, _M), jnp.float32)
    pos_emb = pos_emb.at[:, ::2].set(jnp.sin(arg[:, ::2]))
    pos_emb = pos_emb.at[:, 1::2].set(jnp.cos(arg[:, 1::2]))
    return pos_emb


def kernel(input, emb):
    B, C = input.shape
    assert C == _C and emb.shape[1] == _M

    info = plsc.get_sparse_core_info()
    NC, NS = info.num_cores, info.num_subcores
    NW = NC * NS
    assert B % NW == 0 and B // NW == _BW

    idx_t = input.T
    pos = _positional_embedding(C)

    mesh = plsc.VectorSubcoreMesh(core_axis_name="c", subcore_axis_name="s")

    @pl.kernel(
        out_type=jax.ShapeDtypeStruct((_C, B, _M), jnp.float32),
        mesh=mesh,
        compiler_params=pltpu.CompilerParams(use_tc_tiling_on_sc=False),
        scratch_types=[
            pltpu.VMEM((2, _MC, _BW), jnp.int32),
            pltpu.VMEM((2, _MC, _BW, _M), jnp.float32),
            pltpu.VMEM((_C, _M), jnp.float32),
            pltpu.SemaphoreType.DMA,
            pltpu.SemaphoreType.DMA,
            pltpu.SemaphoreType.DMA,
            pltpu.SemaphoreType.DMA,
        ],
    )
    def run(idx_hbm, emb_hbm, pos_hbm, out_hbm, idx_v, rows_v, pos_v,
            gsem0, gsem1, osem0, osem1):
        gsem = (gsem0, gsem1)
        osem = (osem0, osem1)
        w = lax.axis_index("s") * NC + lax.axis_index("c")
        bw = w * _BW

        def start(g, b):
            c0 = g * _MC
            pltpu.sync_copy(idx_hbm.at[pl.ds(c0, _MC), pl.ds(bw, _BW)],
                            idx_v.at[b])
            for j in range(_MC):
                pltpu.async_copy(emb_hbm.at[idx_v.at[b, j]],
                                 rows_v.at[b, j], gsem[b])

        def wait_gather(b):
            for j in range(_MC):
                pltpu.make_async_copy(emb_hbm.at[pl.ds(0, _BW)],
                                      rows_v.at[b, j], gsem[b]).wait()

        def wait_out(b):
            pltpu.make_async_copy(
                rows_v.at[b],
                out_hbm.at[pl.ds(0, _MC), pl.ds(0, _BW)], osem[b]).wait()

        def add_pos(g, b):
            c0 = g * _MC
            for j in range(_MC):
                pv = [pos_v[c0 + j, pl.ds(16 * q, 16)] for q in range(_M // 16)]

                @pl.loop(0, _BW)
                def _(r):
                    for q in range(_M // 16):
                        plsc.addupdate(
                            rows_v.at[b, j, r, pl.ds(16 * q, 16)], pv[q])

        pltpu.sync_copy(pos_hbm, pos_v)
        start(0, 0)
        start(1, 1)

        @pl.loop(0, _G, step=2)
        def _(t):
            for b in range(2):
                g = t + b
                wait_gather(b)
                add_pos(g, b)
                pltpu.async_copy(
                    rows_v.at[b],
                    out_hbm.at[pl.ds(g * _MC, _MC), pl.ds(bw, _BW)],
                    osem[b])

                @pl.when(g + 2 < _G)
                def _():
                    wait_out(b)
                    start(g + 2, b)

        wait_out(0)
        wait_out(1)

    out = run(idx_t, emb, pos)
    return out.transpose(1, 0, 2)


# speedup vs baseline: 1.5694x; 1.3088x over previous
import jax
import jax.numpy as jnp
from jax import lax
from jax.experimental import pallas as pl
from jax.experimental.pallas import tpu as pltpu
from jax.experimental.pallas import tpu_sc as plsc

_M = 64
_C = 200
_BW = 128
_PC = 8
_NCK = _C // _PC


def _positional_embedding(num_positions):
    pos = jnp.arange(num_positions)
    denom = 10000 ** jnp.linspace(0, 1, _M)
    arg = jnp.expand_dims(pos, 1) / jnp.expand_dims(denom, 0)
    pos_emb = jnp.zeros((num_positions, _M), jnp.float32)
    pos_emb = pos_emb.at[:, ::2].set(jnp.sin(arg[:, ::2]))
    pos_emb = pos_emb.at[:, 1::2].set(jnp.cos(arg[:, 1::2]))
    return pos_emb


def kernel(input, emb):
    B, C = input.shape
    assert C == _C and emb.shape[1] == _M

    info = plsc.get_sparse_core_info()
    NC, NS = info.num_cores, info.num_subcores
    NW = NC * NS
    assert B % NW == 0 and B // NW == _BW

    idx_t = input.T
    emb_t = emb.T
    T = emb.shape[0]
    NPAIR = T // 2
    NBLK = T // 128
    TAIL = T - NBLK * 128
    pos = _positional_embedding(C)
    posx = jnp.broadcast_to(pos[:, :, None], (_C, _M, 16)).reshape(-1)

    mesh = plsc.VectorSubcoreMesh(core_axis_name="c", subcore_axis_name="s")

    @pl.kernel(
        out_type=jax.ShapeDtypeStruct((NPAIR, 2 * _M), jnp.float32),
        mesh=mesh,
        compiler_params=pltpu.CompilerParams(use_tc_tiling_on_sc=True,
                                             needs_layout_passes=False),
        scratch_types=[
            pltpu.VMEM((2, _M, 128), jnp.float32),
            pltpu.VMEM((2, _M, 128), jnp.float32),
            pltpu.VMEM((_M * 130,), jnp.float32),
            pltpu.VMEM((_M * 130,), jnp.float32),
            pltpu.SemaphoreType.DMA,
            pltpu.SemaphoreType.DMA,
            pltpu.SemaphoreType.DMA,
            pltpu.SemaphoreType.DMA,
        ],
    )
    def stage(embt_hbm, tail_hbm, pairs_hbm, x_v, y_v, y1a_v, y1b_v,
              i0, i1, o0, o1):
        y1 = (y1a_v, y1b_v)
        isem = (i0, i1)
        osem = (o0, o1)
        w = lax.axis_index("s") * NC + lax.axis_index("c")
        iota16 = lax.iota(jnp.int32, 16)
        bases = []
        for q in range(8):
            c = iota16 + 16 * q
            bases.append((c >> 1) * 130 + (c & 1) * 65)

        def t_of(n):
            return w + NW * n

        def in_start(n, k):
            pltpu.async_copy(embt_hbm.at[:, pl.ds(t_of(n) * 128, 128)],
                             x_v.at[k], isem[k])

        def in_wait(k):
            pltpu.make_async_copy(embt_hbm.at[:, pl.ds(0, 128)], x_v.at[k],
                                  isem[k]).wait()

        def out_start(n, k):
            pltpu.async_copy(y_v.at[k],
                             pairs_hbm.at[pl.ds(t_of(n) * 64, 64)], osem[k])

        def out_wait(k):
            pltpu.make_async_copy(y_v.at[k], pairs_hbm.at[pl.ds(0, 64)],
                                  osem[k]).wait()

        def transpose_blk(k):
            @plsc.parallel_loop(0, _M, unroll=4)
            def _(j):
                for q in range(8):
                    v = x_v[k, j, pl.ds(16 * q, 16)]
                    plsc.store_scatter(y1[k], [bases[q] + j], v)

            @plsc.parallel_loop(0, _M, unroll=4)
            def _(r):
                for h in range(2):
                    for q2 in range(4):
                        off = r * 130 + 65 * h + 16 * q2
                        val = plsc.load_gather(y1[k], [off + iota16])
                        y_v[k, r, pl.ds(64 * h + 16 * q2, 16)] = val

        nfull = NBLK // NW
        in_start(0, 0)
        in_start(1, 1)

        @pl.loop(0, nfull, step=2)
        def _(n):
            for k in range(2):
                j = n + k
                in_wait(k)

                @pl.when(j >= 2)
                def _():
                    out_wait(k)

                transpose_blk(k)
                out_start(j, k)

                @pl.when(t_of(j + 2) < NBLK)
                def _():
                    in_start(j + 2, k)

        extra = NBLK - nfull * NW
        if extra:
            @pl.when(w < extra)
            def _():
                in_wait(0)
                out_wait(0)
                transpose_blk(0)
                out_start(nfull, 0)

        out_wait(0)
        out_wait(1)

        if TAIL:
            @pl.when(w == 0)
            def _():
                pltpu.sync_copy(tail_hbm, y_v.at[0, pl.ds(0, TAIL // 2)])
                pltpu.sync_copy(y_v.at[0, pl.ds(0, TAIL // 2)],
                                pairs_hbm.at[pl.ds(NBLK * 64, TAIL // 2)])

    @pl.kernel(
        out_type=jax.ShapeDtypeStruct((_C, _M, B), jnp.float32),
        mesh=mesh,
        compiler_params=pltpu.CompilerParams(use_tc_tiling_on_sc=True,
                                             needs_layout_passes=False),
        scratch_types=[
            pltpu.VMEM((2, _PC, _BW), jnp.int32),
            pltpu.VMEM((2, _PC, _BW), jnp.int32),
            pltpu.VMEM((2, _PC, _BW), jnp.int32),
            pltpu.VMEM((2, _BW, 2 * _M), jnp.float32),
            pltpu.VMEM((2, _M, _BW), jnp.float32),
            pltpu.VMEM((_PC * _M * 16,), jnp.float32),
            pltpu.VMEM((_PC * _M * 16,), jnp.float32),
            pltpu.SemaphoreType.DMA,
            pltpu.SemaphoreType.DMA,
            pltpu.SemaphoreType.DMA,
            pltpu.SemaphoreType.DMA,
            pltpu.SemaphoreType.DMA,
            pltpu.SemaphoreType.DMA,
        ],
    )
    def run(idx_hbm, tab_hbm, posx_hbm, out_hbm, idx_v, pidx_v, hv64_v,
            rows_v, t_v, posx0_v, posx1_v, isem, psem, gsem0, gsem1,
            osem0, osem1):
        gsem = (gsem0, gsem1)
        osem = (osem0, osem1)
        posx_v = (posx0_v, posx1_v)
        w = lax.axis_index("s") * NC + lax.axis_index("c")
        bw = w * _BW
        iotas = [lax.iota(jnp.int32, 16) + 16 * g for g in range(_BW // 16)]

        def idx_fetch(ck, s):
            pltpu.async_copy(idx_hbm.at[pl.ds(ck * _PC, _PC), pl.ds(bw, _BW)],
                             idx_v.at[s], isem)

        def idx_wait(s):
            pltpu.make_async_copy(idx_hbm.at[pl.ds(0, _PC), pl.ds(0, _BW)],
                                  idx_v.at[s], isem).wait()

        def posx_fetch(ck, s):
            pltpu.async_copy(posx_hbm.at[pl.ds(ck * _PC * _M * 16,
                                               _PC * _M * 16)],
                             posx_v[s], psem)

        def posx_wait(s):
            pltpu.make_async_copy(posx_hbm.at[pl.ds(0, _PC * _M * 16)],
                                  posx_v[s], psem).wait()

        def derive(s):
            for p in range(_PC):
                for g in range(_BW // 16):
                    v = idx_v[s, p, pl.ds(16 * g, 16)]
                    pidx_v[s, p, pl.ds(16 * g, 16)] = v >> 1
                    hv64_v[s, p, pl.ds(16 * g, 16)] = (v & 1) << 6

        def gather_start(s, p, b):
            pltpu.async_copy(tab_hbm.at[pidx_v.at[s, p]], rows_v.at[b],
                             gsem[b])

        def gather_wait(b):
            pltpu.make_async_copy(tab_hbm.at[pl.ds(0, _BW)], rows_v.at[b],
                                  gsem[b]).wait()

        def out_start(c, b):
            pltpu.async_copy(t_v.at[b], out_hbm.at[c, :, pl.ds(bw, _BW)],
                             osem[b])

        def out_wait(b):
            pltpu.make_async_copy(t_v.at[b], out_hbm.at[0, :, pl.ds(0, _BW)],
                                  osem[b]).wait()

        def transpose_add(s, p, b):
            hv = [hv64_v[s, p, pl.ds(16 * g, 16)] for g in range(_BW // 16)]

            @plsc.parallel_loop(0, _M, unroll=8)
            def _(f):
                pv = posx_v[s][pl.ds((p * _M + f) * 16, 16)]
                for g in range(_BW // 16):
                    val = plsc.load_gather(rows_v.at[b], [iotas[g], hv[g] + f])
                    t_v[b, f, pl.ds(16 * g, 16)] = val + pv

        idx_fetch(0, 0)
        posx_fetch(0, 0)
        idx_wait(0)
        posx_wait(0)
        derive(0)
        idx_fetch(1, 1)
        posx_fetch(1, 1)
        gather_start(0, 0, 0)
        gather_start(0, 1, 1)

        def chunk_body(ct, cb):
            @pl.when(ct + 1 < _NCK)
            def _():
                idx_wait(1 - cb)
                posx_wait(1 - cb)
                derive(1 - cb)

            for p in range(_PC):
                c = ct * _PC + p
                b = p % 2
                gather_wait(b)

                @pl.when(c >= 2)
                def _():
                    out_wait(b)

                transpose_add(cb, p, b)
                out_start(c, b)
                if p < _PC - 2:
                    gather_start(cb, p + 2, b)
                else:
                    @pl.when(ct + 1 < _NCK)
                    def _():
                        gather_start(1 - cb, p + 2 - _PC, b)

            @pl.when(ct + 2 < _NCK)
            def _():
                idx_fetch(ct + 2, cb)
                posx_fetch(ct + 2, cb)

        @pl.loop(0, _NCK - 1, step=2)
        def _(t):
            chunk_body(t, 0)
            chunk_body(t + 1, 1)

        chunk_body(_NCK - 1, 0)
        out_wait(0)
        out_wait(1)

    tail_rows = emb[NBLK * 128:].reshape(TAIL // 2, 2 * _M) if TAIL else None
    table = stage(emb_t, tail_rows)
    out = run(idx_t, table, posx)
    return out.transpose(2, 0, 1)


# speedup vs baseline: 2.6454x; 1.6856x over previous
import jax
import jax.numpy as jnp
from jax import lax
from jax.experimental import pallas as pl
from jax.experimental.pallas import tpu as pltpu
from jax.experimental.pallas import tpu_sc as plsc

_M = 64
_C = 200
_BW = 128
_PC = 8
_NCK = _C // _PC


def _positional_embedding(num_positions):
    pos = jnp.arange(num_positions)
    denom = 10000 ** jnp.linspace(0, 1, _M)
    arg = jnp.expand_dims(pos, 1) / jnp.expand_dims(denom, 0)
    pos_emb = jnp.zeros((num_positions, _M), jnp.float32)
    pos_emb = pos_emb.at[:, ::2].set(jnp.sin(arg[:, ::2]))
    pos_emb = pos_emb.at[:, 1::2].set(jnp.cos(arg[:, 1::2]))
    return pos_emb


def kernel(input, emb):
    B, C = input.shape
    assert C == _C and emb.shape[1] == _M

    info = plsc.get_sparse_core_info()
    NC, NS = info.num_cores, info.num_subcores
    NW = NC * NS
    assert B % NW == 0 and B // NW == _BW

    idx_t = input.T
    emb_t = emb.T
    T = emb.shape[0]
    NPAIR = T // 2
    NBLK = T // 128
    TAIL = T - NBLK * 128
    pos = _positional_embedding(C)
    posx = jnp.broadcast_to(pos[:, :, None], (_C, _M, 16)).reshape(-1)

    mesh = plsc.VectorSubcoreMesh(core_axis_name="c", subcore_axis_name="s")

    @pl.kernel(
        out_type=jax.ShapeDtypeStruct((NPAIR, 2 * _M), jnp.float32),
        mesh=mesh,
        compiler_params=pltpu.CompilerParams(use_tc_tiling_on_sc=True,
                                             needs_layout_passes=False),
        scratch_types=[
            pltpu.VMEM((2, _M, 128), jnp.float32),
            pltpu.VMEM((2, _M, 128), jnp.float32),
            pltpu.VMEM((_M * 130,), jnp.float32),
            pltpu.VMEM((_M * 130,), jnp.float32),
            pltpu.SemaphoreType.DMA,
            pltpu.SemaphoreType.DMA,
            pltpu.SemaphoreType.DMA,
            pltpu.SemaphoreType.DMA,
        ],
    )
    def stage(embt_hbm, tail_hbm, pairs_hbm, x_v, y_v, y1a_v, y1b_v,
              i0, i1, o0, o1):
        y1 = (y1a_v, y1b_v)
        isem = (i0, i1)
        osem = (o0, o1)
        w = lax.axis_index("s") * NC + lax.axis_index("c")
        iota16 = lax.iota(jnp.int32, 16)
        bases = []
        for q in range(8):
            c = iota16 + 16 * q
            bases.append((c >> 1) * 130 + (c & 1) * 65)

        def t_of(n):
            return w + NW * n

        def in_start(n, k):
            pltpu.async_copy(embt_hbm.at[:, pl.ds(t_of(n) * 128, 128)],
                             x_v.at[k], isem[k])

        def in_wait(k):
            pltpu.make_async_copy(embt_hbm.at[:, pl.ds(0, 128)], x_v.at[k],
                                  isem[k]).wait()

        def out_start(n, k):
            pltpu.async_copy(y_v.at[k],
                             pairs_hbm.at[pl.ds(t_of(n) * 64, 64)], osem[k])

        def out_wait(k):
            pltpu.make_async_copy(y_v.at[k], pairs_hbm.at[pl.ds(0, 64)],
                                  osem[k]).wait()

        def transpose_blk(k):
            @plsc.parallel_loop(0, _M, unroll=4)
            def _(j):
                for q in range(8):
                    v = x_v[k, j, pl.ds(16 * q, 16)]
                    plsc.store_scatter(y1[k], [bases[q] + j], v)

            @plsc.parallel_loop(0, _M, unroll=4)
            def _(r):
                for h in range(2):
                    for q2 in range(4):
                        off = r * 130 + 65 * h + 16 * q2
                        val = plsc.load_gather(y1[k], [off + iota16])
                        y_v[k, r, pl.ds(64 * h + 16 * q2, 16)] = val

        nfull = NBLK // NW
        in_start(0, 0)
        in_start(1, 1)

        @pl.loop(0, nfull, step=2)
        def _(n):
            for k in range(2):
                j = n + k
                in_wait(k)

                @pl.when(j >= 2)
                def _():
                    out_wait(k)

                transpose_blk(k)
                out_start(j, k)

                @pl.when(t_of(j + 2) < NBLK)
                def _():
                    in_start(j + 2, k)

        extra = NBLK - nfull * NW
        if extra:
            @pl.when(w < extra)
            def _():
                in_wait(0)
                out_wait(0)
                transpose_blk(0)
                out_start(nfull, 0)

        out_wait(0)
        out_wait(1)

        if TAIL:
            @pl.when(w == 0)
            def _():
                pltpu.sync_copy(tail_hbm, y_v.at[0, pl.ds(0, TAIL // 2)])
                pltpu.sync_copy(y_v.at[0, pl.ds(0, TAIL // 2)],
                                pairs_hbm.at[pl.ds(NBLK * 64, TAIL // 2)])

    @pl.kernel(
        out_type=jax.ShapeDtypeStruct((_C, _M, B), jnp.float32),
        mesh=mesh,
        compiler_params=pltpu.CompilerParams(use_tc_tiling_on_sc=True,
                                             needs_layout_passes=False),
        scratch_types=[
            pltpu.VMEM((2, _PC, _BW), jnp.int32),
            pltpu.VMEM((2, _PC, _BW), jnp.int32),
            pltpu.VMEM((2, _PC, _BW), jnp.int32),
            pltpu.VMEM((2, _BW, 2 * _M), jnp.float32),
            pltpu.VMEM((2, _M, _BW), jnp.float32),
            pltpu.VMEM((_PC * _M * 16,), jnp.float32),
            pltpu.VMEM((_PC * _M * 16,), jnp.float32),
            pltpu.VMEM((_BW * 129,), jnp.float32),
            pltpu.VMEM((_BW * 129,), jnp.float32),
            pltpu.SemaphoreType.DMA,
            pltpu.SemaphoreType.DMA,
            pltpu.SemaphoreType.DMA,
            pltpu.SemaphoreType.DMA,
            pltpu.SemaphoreType.DMA,
            pltpu.SemaphoreType.DMA,
        ],
    )
    def run(idx_hbm, tab_hbm, posx_hbm, out_hbm, idx_v, pidx_v, hv64_v,
            rows_v, t_v, posx0_v, posx1_v, ua_v, ub_v, isem, psem,
            gsem0, gsem1, osem0, osem1):
        gsem = (gsem0, gsem1)
        osem = (osem0, osem1)
        posx_v = (posx0_v, posx1_v)
        u = (ua_v, ub_v)
        w = lax.axis_index("s") * NC + lax.axis_index("c")
        bw = w * _BW
        iota16 = lax.iota(jnp.int32, 16)
        iotas = [iota16 + 16 * g for g in range(_BW // 16)]
        cbases = [iotas[q] * 129 for q in range(2 * _M // 16)]

        def idx_fetch(ck, s):
            pltpu.async_copy(idx_hbm.at[pl.ds(ck * _PC, _PC), pl.ds(bw, _BW)],
                             idx_v.at[s], isem)

        def idx_wait(s):
            pltpu.make_async_copy(idx_hbm.at[pl.ds(0, _PC), pl.ds(0, _BW)],
                                  idx_v.at[s], isem).wait()

        def posx_fetch(ck, s):
            pltpu.async_copy(posx_hbm.at[pl.ds(ck * _PC * _M * 16,
                                               _PC * _M * 16)],
                             posx_v[s], psem)

        def posx_wait(s):
            pltpu.make_async_copy(posx_hbm.at[pl.ds(0, _PC * _M * 16)],
                                  posx_v[s], psem).wait()

        def derive(s):
            for p in range(_PC):
                for g in range(_BW // 16):
                    v = idx_v[s, p, pl.ds(16 * g, 16)]
                    pidx_v[s, p, pl.ds(16 * g, 16)] = v >> 1
                    hv64_v[s, p, pl.ds(16 * g, 16)] = (v & 1) << 6

        def gather_start(s, p, b):
            pltpu.async_copy(tab_hbm.at[pidx_v.at[s, p]], rows_v.at[b],
                             gsem[b])

        def gather_wait(b):
            pltpu.make_async_copy(tab_hbm.at[pl.ds(0, _BW)], rows_v.at[b],
                                  gsem[b]).wait()

        def out_start(c, b):
            pltpu.async_copy(t_v.at[b], out_hbm.at[c, :, pl.ds(bw, _BW)],
                             osem[b])

        def out_wait(b):
            pltpu.make_async_copy(t_v.at[b], out_hbm.at[0, :, pl.ds(0, _BW)],
                                  osem[b]).wait()

        def transpose_add(s, p, b):
            @plsc.parallel_loop(0, _BW, unroll=2)
            def _(r):
                for q in range(2 * _M // 16):
                    v = rows_v[b, r, pl.ds(16 * q, 16)]
                    plsc.store_scatter(u[b], [cbases[q] + r], v)

            ha = [hv64_v[s, p, pl.ds(16 * g, 16)] * 129
                  for g in range(_BW // 16)]

            @plsc.parallel_loop(0, _M, unroll=2)
            def _(f):
                pv = posx_v[s][pl.ds((p * _M + f) * 16, 16)]
                fb = f * 129
                for g in range(_BW // 16):
                    val = plsc.load_gather(
                        u[b], [ha[g] + (fb + 16 * g) + iota16])
                    t_v[b, f, pl.ds(16 * g, 16)] = val + pv

        idx_fetch(0, 0)
        posx_fetch(0, 0)
        idx_wait(0)
        posx_wait(0)
        derive(0)
        idx_fetch(1, 1)
        posx_fetch(1, 1)
        gather_start(0, 0, 0)
        gather_start(0, 1, 1)

        def chunk_body(ct, cb):
            @pl.when(ct + 1 < _NCK)
            def _():
                idx_wait(1 - cb)
                posx_wait(1 - cb)
                derive(1 - cb)

            for p in range(_PC):
                c = ct * _PC + p
                b = p % 2
                gather_wait(b)

                @pl.when(c >= 2)
                def _():
                    out_wait(b)

                transpose_add(cb, p, b)
                out_start(c, b)
                if p < _PC - 2:
                    gather_start(cb, p + 2, b)
                else:
                    @pl.when(ct + 1 < _NCK)
                    def _():
                        gather_start(1 - cb, p + 2 - _PC, b)

            @pl.when(ct + 2 < _NCK)
            def _():
                idx_fetch(ct + 2, cb)
                posx_fetch(ct + 2, cb)

        @pl.loop(0, _NCK - 1, step=2)
        def _(t):
            chunk_body(t, 0)
            chunk_body(t + 1, 1)

        chunk_body(_NCK - 1, 0)
        out_wait(0)
        out_wait(1)

    tail_rows = emb[NBLK * 128:].reshape(TAIL // 2, 2 * _M) if TAIL else None
    table = stage(emb_t, tail_rows)
    out = run(idx_t, table, posx)
    return out.transpose(2, 0, 1)
